# baseline (speedup 1.0000x reference)
import jax
import jax.numpy as jnp
from jax.experimental import pallas as pl
from jax.experimental.pallas import tpu as pltpu


def _shuffle_kernel(pos_ref, emb_ref, hs_ref, mask_ref,
                    out_hs_ref, out_mask_ref,
                    sem_big, sem_ctx):
    batch = out_hs_ref.shape[0]
    n_ctx = emb_ref.shape[1]
    seq = hs_ref.shape[1]

    big = pltpu.make_async_copy(
        hs_ref, out_hs_ref.at[:, pl.ds(n_ctx, seq), :], sem_big)
    big.start()

    p = pos_ref[0]
    ctx_copies = []
    for b in range(batch):
        c = pltpu.make_async_copy(
            emb_ref.at[p], out_hs_ref.at[b, pl.ds(0, n_ctx), :], sem_ctx)
        c.start()
        ctx_copies.append(c)

    out_mask_ref[:, :n_ctx] = jnp.ones_like(out_mask_ref[:, :n_ctx])
    out_mask_ref[:, n_ctx:] = mask_ref[:]

    big.wait()
    for c in ctx_copies:
        c.wait()


def kernel(hidden_states, attention_mask, position, embeddings):
    B, S, H = hidden_states.shape
    T = embeddings.shape[1]
    pos = jnp.asarray(position, dtype=jnp.int32).reshape((1,))

    out_hs, out_mask = pl.pallas_call(
        _shuffle_kernel,
        in_specs=[
            pl.BlockSpec(memory_space=pltpu.SMEM),
            pl.BlockSpec(memory_space=pl.ANY),
            pl.BlockSpec(memory_space=pl.ANY),
            pl.BlockSpec(memory_space=pltpu.VMEM),
        ],
        out_specs=[
            pl.BlockSpec(memory_space=pl.ANY),
            pl.BlockSpec(memory_space=pltpu.VMEM),
        ],
        out_shape=[
            jax.ShapeDtypeStruct((B, T + S, H), hidden_states.dtype),
            jax.ShapeDtypeStruct((B, T + S), attention_mask.dtype),
        ],
        scratch_shapes=[pltpu.SemaphoreType.DMA, pltpu.SemaphoreType.DMA],
    )(pos, embeddings, hidden_states, attention_mask)
    return out_hs, out_mask
